# baseline (speedup 1.0000x reference)
import functools

import jax
import jax.numpy as jnp
from jax import lax
from jax.experimental import pallas as pl
from jax.experimental.pallas import tpu as pltpu
from jax.experimental.pallas import tpu_sc as plsc

N_VOCAB = 1000000
EMBED_DIM = 32
BATCH = 16384

_NC = 2
_NS = 16
_NW = _NC * _NS
_B_PER_W = BATCH // _NW
_CHUNK = 128
_N_CHUNKS = _B_PER_W // _CHUNK


def _gather_body(idx_hbm, table_hbm, out_hbm, idx_v, rows_v, sem):
    wid = lax.axis_index("s") * _NC + lax.axis_index("c")
    base = wid * _B_PER_W
    pltpu.sync_copy(idx_hbm.at[pl.ds(base, _B_PER_W)], idx_v)
    copies = []
    for j in range(_N_CHUNKS):
        copies.append(
            pltpu.async_copy(
                table_hbm.at[idx_v.at[pl.ds(j * _CHUNK, _CHUNK)]],
                rows_v.at[pl.ds(j * _CHUNK, _CHUNK)],
                sem,
            )
        )
    for c in copies:
        c.wait()
    pltpu.sync_copy(rows_v, out_hbm.at[pl.ds(base, _B_PER_W)])


@jax.jit
def _sc_gather(user_id, table):
    mesh = plsc.VectorSubcoreMesh(core_axis_name="c", subcore_axis_name="s")
    return pl.kernel(
        _gather_body,
        out_type=jax.ShapeDtypeStruct((BATCH, EMBED_DIM), jnp.float32),
        mesh=mesh,
        scratch_types=[
            pltpu.VMEM((_B_PER_W,), jnp.int32),
            pltpu.VMEM((_B_PER_W, EMBED_DIM), jnp.float32),
            pltpu.SemaphoreType.DMA,
        ],
        compiler_params=pltpu.CompilerParams(use_tc_tiling_on_sc=False),
    )(user_id, table)


def kernel(user_id, table):
    return _sc_gather(user_id, table)


# speedup vs baseline: 2.1417x; 2.1417x over previous
import functools

import jax
import jax.numpy as jnp
from jax import lax
from jax.experimental import pallas as pl
from jax.experimental.pallas import tpu as pltpu
from jax.experimental.pallas import tpu_sc as plsc

N_VOCAB = 1000000
EMBED_DIM = 32
BATCH = 16384

_NC = 2
_NS = 16
_NW = _NC * _NS
_B_PER_W = BATCH // _NW
_LANES = 128
_CH = 16
_N_CH = _B_PER_W // _CH


def _fire(tab_t_hbm, idx, buf, sem):
    off = pl.multiple_of((idx >> 7) << 7, _LANES)
    for tr in range(EMBED_DIM // 8):
        pltpu.async_copy(
            tab_t_hbm.at[pl.ds(8 * tr, 8), pl.ds(off, _LANES)],
            buf.at[pl.ds(8 * tr, 8), :],
            sem,
        )


def _drain(tab_t_hbm, buf, sem):
    pltpu.make_async_copy(tab_t_hbm.at[:, pl.ds(0, _LANES)], buf, sem).wait()


def _extract(idx, j, buf, cols_v, rows0, rows1):
    lane = jnp.full((16,), idx & (_LANES - 1), jnp.int32)
    col = jnp.full((16,), j, jnp.int32)
    g0 = plsc.load_gather(buf, [rows0, lane])
    g1 = plsc.load_gather(buf, [rows1, lane])
    plsc.store_scatter(cols_v, [rows0, col], g0)
    plsc.store_scatter(cols_v, [rows1, col], g1)


def _gather_body(idx_hbm, tab_t_hbm, out_t_hbm, idx_s, buf0, buf1,
                 cols_v, sem0, sem1):
    wid = lax.axis_index("s") * _NC + lax.axis_index("c")
    base = wid * _B_PER_W
    pltpu.sync_copy(idx_hbm.at[pl.ds(base, _B_PER_W)], idx_s)

    rows0 = lax.iota(jnp.int32, 16)
    rows1 = rows0 + 16
    bufs = (buf0, buf1)
    sems = (sem0, sem1)

    v0 = idx_s[pl.ds(0, _CH)]
    _fire(tab_t_hbm, v0[0], buf0, sem0)
    _fire(tab_t_hbm, v0[1], buf1, sem1)

    def _chunk(k, vcur):
        nk = jnp.minimum(k + 1, _N_CH - 1)
        vnext = idx_s[pl.ds(nk * _CH, _CH)]
        for i in range(_CH):
            j = k * _CH + i
            buf, sem = bufs[i % 2], sems[i % 2]
            _drain(tab_t_hbm, buf, sem)
            _extract(vcur[i], j, buf, cols_v, rows0, rows1)
            if i < _CH - 2:
                _fire(tab_t_hbm, vcur[i + 2], buf, sem)
            else:

                @pl.when(k < _N_CH - 1)
                def _():
                    _fire(tab_t_hbm, vnext[i - (_CH - 2)], buf, sem)

        return vnext

    lax.fori_loop(0, _N_CH, _chunk, v0, unroll=False)

    pltpu.sync_copy(cols_v, out_t_hbm.at[:, pl.ds(base, _B_PER_W)])


@jax.jit
def _sc_gather(user_id, table_t):
    mesh = plsc.VectorSubcoreMesh(core_axis_name="c", subcore_axis_name="s")
    return pl.kernel(
        _gather_body,
        out_type=jax.ShapeDtypeStruct((EMBED_DIM, BATCH), jnp.float32),
        mesh=mesh,
        scratch_types=[
            pltpu.VMEM((_B_PER_W,), jnp.int32),
            pltpu.VMEM((EMBED_DIM, _LANES), jnp.float32),
            pltpu.VMEM((EMBED_DIM, _LANES), jnp.float32),
            pltpu.VMEM((EMBED_DIM, _B_PER_W), jnp.float32),
            pltpu.SemaphoreType.DMA,
            pltpu.SemaphoreType.DMA,
        ],
        compiler_params=pltpu.CompilerParams(needs_layout_passes=False),
    )(user_id, table_t)


def kernel(user_id, table):
    out_t = _sc_gather(user_id, table.T)
    return out_t.T


# speedup vs baseline: 4.2146x; 1.9679x over previous
import functools

import jax
import jax.numpy as jnp
from jax import lax
from jax.experimental import pallas as pl
from jax.experimental.pallas import tpu as pltpu
from jax.experimental.pallas import tpu_sc as plsc

N_VOCAB = 1000000
EMBED_DIM = 32
BATCH = 16384

_NC = 2
_NS = 16
_NW = _NC * _NS
_B_PER_W = BATCH // _NW
_LANES = 128
_CH = 16
_N_CH = _B_PER_W // _CH
_DEPTH = 8


def _fire(tab_t_hbm, idx, buf, sem):
    off = pl.multiple_of((idx >> 7) << 7, _LANES)
    pltpu.async_copy(tab_t_hbm.at[:, pl.ds(off, _LANES)], buf, sem)


def _drain(tab_t_hbm, buf, sem):
    pltpu.make_async_copy(tab_t_hbm.at[:, pl.ds(0, _LANES)], buf, sem).wait()


def _extract(idx, j, buf, cols_v, rows0, rows1):
    lane = jnp.full((16,), idx & (_LANES - 1), jnp.int32)
    col = jnp.full((16,), j, jnp.int32)
    g0 = plsc.load_gather(buf, [rows0, lane])
    g1 = plsc.load_gather(buf, [rows1, lane])
    plsc.store_scatter(cols_v, [rows0, col], g0)
    plsc.store_scatter(cols_v, [rows1, col], g1)


def _gather_body(idx_hbm, tab_t_hbm, out_t_hbm, idx_s,
                 b0, b1, b2, b3, b4, b5, b6, b7, cols_v,
                 s0, s1, s2, s3, s4, s5, s6, s7):
    wid = lax.axis_index("s") * _NC + lax.axis_index("c")
    base = wid * _B_PER_W
    pltpu.sync_copy(idx_hbm.at[pl.ds(base, _B_PER_W)], idx_s)

    rows0 = lax.iota(jnp.int32, 16)
    rows1 = rows0 + 16
    bufs = (b0, b1, b2, b3, b4, b5, b6, b7)
    sems = (s0, s1, s2, s3, s4, s5, s6, s7)

    v0 = idx_s[pl.ds(0, _CH)]
    for i in range(_DEPTH):
        _fire(tab_t_hbm, v0[i], bufs[i], sems[i])

    def _chunk(k, vcur):
        nk = jnp.minimum(k + 1, _N_CH - 1)
        vnext = idx_s[pl.ds(nk * _CH, _CH)]
        for i in range(_CH):
            j = k * _CH + i
            buf, sem = bufs[i % _DEPTH], sems[i % _DEPTH]
            _drain(tab_t_hbm, buf, sem)
            _extract(vcur[i], j, buf, cols_v, rows0, rows1)
            if i < _CH - _DEPTH:
                _fire(tab_t_hbm, vcur[i + _DEPTH], buf, sem)
            else:

                @pl.when(k < _N_CH - 1)
                def _():
                    _fire(tab_t_hbm, vnext[i - (_CH - _DEPTH)], buf, sem)

        return vnext

    lax.fori_loop(0, _N_CH, _chunk, v0, unroll=False)

    pltpu.sync_copy(cols_v, out_t_hbm.at[:, pl.ds(base, _B_PER_W)])


@jax.jit
def _sc_gather(user_id, table_t):
    mesh = plsc.VectorSubcoreMesh(core_axis_name="c", subcore_axis_name="s")
    return pl.kernel(
        _gather_body,
        out_type=jax.ShapeDtypeStruct((EMBED_DIM, BATCH), jnp.float32),
        mesh=mesh,
        scratch_types=(
            [pltpu.VMEM((_B_PER_W,), jnp.int32)]
            + [pltpu.VMEM((EMBED_DIM, _LANES), jnp.float32)] * _DEPTH
            + [pltpu.VMEM((EMBED_DIM, _B_PER_W), jnp.float32)]
            + [pltpu.SemaphoreType.DMA] * _DEPTH
        ),
        compiler_params=pltpu.CompilerParams(needs_layout_passes=False),
    )(user_id, table_t)


def kernel(user_id, table):
    out_t = _sc_gather(user_id, table.T)
    return out_t.T
